# Initial kernel scaffold; baseline (speedup 1.0000x reference)
import jax
import jax.numpy as jnp
from jax.experimental import pallas as pl


def kernel(x, bw1, b1row, bw2, b2row, w1p, b1p, w2p, b2p):
    raise NotImplementedError("write your pallas kernel here")


# speedup vs baseline: 3.8387x; 3.8387x over previous
import functools

import jax
import jax.numpy as jnp
from jax.experimental import pallas as pl
from jax.experimental.pallas import tpu as pltpu


def _conv_stack_kernel(x_ref, w1_ref, b1_ref, w2_ref, b2_ref, o_ref, *,
                       h, w, cin, ch):
    f32 = jnp.float32
    bf16 = jnp.bfloat16
    B = x_ref.shape[0]
    wcin = w * cin
    wch = w * ch
    h2 = h // 2
    wp = w // 2
    w2c = wp * ch
    M1 = B * h

    X = x_ref[...].reshape(M1, wcin)
    rows = jax.lax.broadcasted_iota(jnp.int32, (M1, wcin), 0)
    z1 = jnp.zeros((1, wcin), f32)
    Xd = jnp.where(rows % h == 0, 0.0, jnp.concatenate([z1, X[:-1]], axis=0))
    Xu = jnp.where(rows % h == h - 1, 0.0,
                   jnp.concatenate([X[1:], z1], axis=0))
    X3 = jnp.concatenate([Xd, X, Xu], axis=1).astype(bf16)
    acc1 = jnp.dot(X3, w1_ref[...], preferred_element_type=f32)
    acc1 = jnp.maximum(acc1 + b1_ref[...], 0.0)

    rm = jnp.max(acc1.reshape(M1 // 2, 2, wch), axis=1)
    p1 = jnp.concatenate(
        [jnp.maximum(rm[:, (2 * p) * ch:(2 * p + 1) * ch],
                     rm[:, (2 * p + 1) * ch:(2 * p + 2) * ch])
         for p in range(wp)], axis=1)

    M2 = M1 // 2
    rows2 = jax.lax.broadcasted_iota(jnp.int32, (M2, w2c), 0)
    z2 = jnp.zeros((1, w2c), f32)
    Pd = jnp.where(rows2 % h2 == 0, 0.0,
                   jnp.concatenate([z2, p1[:-1]], axis=0))
    Pu = jnp.where(rows2 % h2 == h2 - 1, 0.0,
                   jnp.concatenate([p1[1:], z2], axis=0))
    P3 = jnp.concatenate([Pd, p1, Pu], axis=1).astype(bf16)
    acc2 = jnp.dot(P3, w2_ref[...], preferred_element_type=f32)
    acc2 = jnp.maximum(acc2 + b2_ref[...], 0.0)

    rm2 = jnp.max(acc2.reshape(M2 // 2, 2, w2c), axis=1)
    w4 = wp // 2
    p2 = jnp.concatenate(
        [jnp.maximum(rm2[:, (2 * p) * ch:(2 * p + 1) * ch],
                     rm2[:, (2 * p + 1) * ch:(2 * p + 2) * ch])
         for p in range(w4)], axis=1)
    o_ref[...] = p2.reshape(B, h // 4, w4 * ch).astype(o_ref.dtype)


def _mlp_kernel(x_ref, w1_ref, b1_ref, w2_ref, b2_ref, o_ref):
    f32 = jnp.float32
    hid = jnp.dot(x_ref[...], w1_ref[...], preferred_element_type=f32)
    hid = jnp.maximum(hid + b1_ref[...], 0.0).astype(jnp.bfloat16)
    out = jnp.dot(hid, w2_ref[...], preferred_element_type=f32) + b2_ref[...]
    o_ref[...] = out


def _forward(x, bw1, b1row, bw2, b2row, w1p, b1p, w2p, b2p, *, num_classes):
    n, cin, h, w = x.shape
    wch = b1row.shape[1]
    ch = wch // w
    w2c = b2row.shape[1]
    h4, w4 = h // 4, w // 4
    hp = w1p.shape[1]
    cp = w2p.shape[1]
    bf16 = jnp.bfloat16

    xt = jnp.transpose(x, (0, 2, 1, 3)).reshape(n, h, cin * w)
    w1cat = jnp.transpose(bw1, (1, 0, 2, 3)).reshape(3 * cin * w, wch).astype(bf16)
    w2cat = bw2.reshape(3 * w2c, w2c).astype(bf16)

    B = next(b for b in (16, 8, 4, 2, 1) if n % b == 0)
    feats = pl.pallas_call(
        functools.partial(_conv_stack_kernel, h=h, w=w, cin=cin, ch=ch),
        out_shape=jax.ShapeDtypeStruct((n, h4, w4 * ch), bf16),
        grid=(n // B,),
        in_specs=[
            pl.BlockSpec((B, h, cin * w), lambda i: (i, 0, 0)),
            pl.BlockSpec((3 * cin * w, wch), lambda i: (0, 0)),
            pl.BlockSpec((1, wch), lambda i: (0, 0)),
            pl.BlockSpec((3 * w2c, w2c), lambda i: (0, 0)),
            pl.BlockSpec((1, w2c), lambda i: (0, 0)),
        ],
        out_specs=pl.BlockSpec((B, h4, w4 * ch), lambda i: (i, 0, 0)),
        compiler_params=pltpu.CompilerParams(
            dimension_semantics=("parallel",)),
    )(xt, w1cat, b1row, w2cat, b2row)

    flat = feats.reshape(n, h4 * w4 * ch)
    mt = 128 if n % 128 == 0 else n
    logits = pl.pallas_call(
        _mlp_kernel,
        out_shape=jax.ShapeDtypeStruct((n, cp), jnp.float32),
        grid=(n // mt,),
        in_specs=[
            pl.BlockSpec((mt, h4 * w4 * ch), lambda i: (i, 0)),
            pl.BlockSpec((h4 * w4 * ch, hp), lambda i: (0, 0)),
            pl.BlockSpec((1, hp), lambda i: (0, 0)),
            pl.BlockSpec((hp, cp), lambda i: (0, 0)),
            pl.BlockSpec((1, cp), lambda i: (0, 0)),
        ],
        out_specs=pl.BlockSpec((mt, cp), lambda i: (i, 0)),
        compiler_params=pltpu.CompilerParams(
            dimension_semantics=("parallel",)),
    )(flat, w1p.astype(bf16), b1p, w2p.astype(bf16), b2p)
    return {"out": logits[:, :num_classes]}


def kernel(x, bw1, b1row, bw2, b2row, w1p, b1p, w2p, b2p):
    return _forward(x, bw1, b1row, bw2, b2row, w1p, b1p, w2p, b2p,
                    num_classes=100)
